# baseline (speedup 1.0000x reference)
import functools

import jax
import jax.numpy as jnp
from jax.experimental import pallas as pl
from jax.experimental.pallas import tpu as pltpu


def _chamfer_body(q_ref, r_ref, out_ref, qaug_scratch, rowmin_scratch, *,
                  n_chunks, q_total):
    i = pl.program_id(0)

    @pl.when(i == 0)
    def _prep():
        q = q_ref[:, :]
        d = q.shape[1]
        qaug_scratch[:, :d] = (q * -2.0).astype(jnp.bfloat16)
        qaug_scratch[:, d:] = jnp.ones_like(qaug_scratch[:, d:])

    r_c = r_ref[:, :]
    raug_c = jnp.concatenate([r_c, r_c * r_c], axis=1).astype(jnp.bfloat16)

    t_c = jax.lax.dot_general(
        qaug_scratch[:, :],
        raug_c,
        dimension_numbers=(((1,), (1,)), ((), ())),
        preferred_element_type=jnp.float32,
    )

    tq = t_c.shape[0]
    m_c = jnp.min(t_c.reshape(tq // 8, 8, t_c.shape[1]), axis=2)

    @pl.when(i == 0)
    def _init():
        rowmin_scratch[:, :] = m_c

    @pl.when(i > 0)
    def _acc():
        rowmin_scratch[:, :] = jnp.minimum(rowmin_scratch[:, :], m_c)

    @pl.when(i == n_chunks - 1)
    def _finish():
        q = q_ref[:, :]
        total = jnp.sum(rowmin_scratch[:, :]) + jnp.sum(q * q)
        out_ref[:, :] = total.reshape(1, 1) / q_total


def kernel(query, ref):
    q_total, d = query.shape
    r_total, _ = ref.shape

    chunk = 2048 if r_total % 2048 == 0 else r_total
    n_chunks = r_total // chunk

    body = functools.partial(_chamfer_body, n_chunks=n_chunks,
                             q_total=float(q_total))
    out = pl.pallas_call(
        body,
        grid=(n_chunks,),
        in_specs=[
            pl.BlockSpec((q_total, d), lambda i: (0, 0)),
            pl.BlockSpec((chunk, d), lambda i: (i, 0)),
        ],
        out_specs=pl.BlockSpec((1, 1), lambda i: (0, 0)),
        out_shape=jax.ShapeDtypeStruct((1, 1), jnp.float32),
        scratch_shapes=[
            pltpu.VMEM((q_total, 2 * d), jnp.bfloat16),
            pltpu.VMEM((q_total // 8, 8), jnp.float32),
        ],
        compiler_params=pltpu.CompilerParams(
            vmem_limit_bytes=128 * 1024 * 1024),
    )(query, ref)
    return out[0, 0]


# speedup vs baseline: 1.0439x; 1.0439x over previous
import functools

import jax
import jax.numpy as jnp
from jax.experimental import pallas as pl
from jax.experimental.pallas import tpu as pltpu


def _chamfer_body(q_ref, r_ref, out_ref, qaug_scratch, rowmin_scratch, *,
                  n_chunks, q_total):
    i = pl.program_id(0)

    @pl.when(i == 0)
    def _prep():
        q = q_ref[:, :]
        d = q.shape[1]
        qaug_scratch[:, :d] = (q * -2.0).astype(jnp.bfloat16)
        qaug_scratch[:, d:] = jnp.ones_like(qaug_scratch[:, d:])

    r_c = r_ref[:, :]
    raug_c = jnp.concatenate([r_c, r_c * r_c], axis=1).astype(jnp.bfloat16)

    t_c = jax.lax.dot_general(
        qaug_scratch[:, :],
        raug_c,
        dimension_numbers=(((1,), (1,)), ((), ())),
        preferred_element_type=jnp.float32,
    )

    tq = t_c.shape[0]
    m_c = jnp.min(t_c.reshape(tq // 8, 8, t_c.shape[1]), axis=2)

    @pl.when(i == 0)
    def _init():
        rowmin_scratch[:, :] = m_c

    @pl.when(i > 0)
    def _acc():
        rowmin_scratch[:, :] = jnp.minimum(rowmin_scratch[:, :], m_c)

    @pl.when(i == n_chunks - 1)
    def _finish():
        q = q_ref[:, :]
        total = jnp.sum(rowmin_scratch[:, :]) + jnp.sum(q * q)
        out_ref[:, :] = total.reshape(1, 1) / q_total


def kernel(query, ref):
    q_total, d = query.shape
    r_total, _ = ref.shape

    chunk = 4096 if r_total % 4096 == 0 else r_total
    n_chunks = r_total // chunk

    body = functools.partial(_chamfer_body, n_chunks=n_chunks,
                             q_total=float(q_total))
    out = pl.pallas_call(
        body,
        grid=(n_chunks,),
        in_specs=[
            pl.BlockSpec((q_total, d), lambda i: (0, 0)),
            pl.BlockSpec((chunk, d), lambda i: (i, 0)),
        ],
        out_specs=pl.BlockSpec((1, 1), lambda i: (0, 0)),
        out_shape=jax.ShapeDtypeStruct((1, 1), jnp.float32),
        scratch_shapes=[
            pltpu.VMEM((q_total, 2 * d), jnp.bfloat16),
            pltpu.VMEM((q_total // 8, 8), jnp.float32),
        ],
        compiler_params=pltpu.CompilerParams(
            vmem_limit_bytes=128 * 1024 * 1024),
    )(query, ref)
    return out[0, 0]
